# baseline (speedup 1.0000x reference)
import functools

import jax
import jax.numpy as jnp
from jax import lax
from jax.experimental import pallas as pl
from jax.experimental.pallas import tpu as pltpu
from jax.experimental.pallas import tpu_sc as plsc

DIM = 1024
SEQ = 8192
BATCH = 2

NUM_WORKERS = 32
ROWS_PER_W = SEQ // NUM_WORKERS
CHUNK = 32
NCHUNK = ROWS_PER_W // CHUNK

ACHUNK = 8
NACHUNK = ROWS_PER_W // ACHUNK
CE = ACHUNK * DIM
DEPTH = 4

BS = 512
NBLK = SEQ // BS


def _sc_gather_body(pe_hbm, pos_hbm, g_hbm, idx_v, buf0, buf1, sem0, sem1):
    wid = lax.axis_index("s") * 2 + lax.axis_index("c")
    base = wid * ROWS_PER_W
    pltpu.sync_copy(pos_hbm.at[pl.ds(base, ROWS_PER_W)], idx_v)
    bufs = (buf0, buf1)
    sems = (sem0, sem1)
    copies = []
    for c in range(NCHUNK):
        copies.append(
            pltpu.async_copy(
                pe_hbm.at[idx_v.at[pl.ds(c * CHUNK, CHUNK)]],
                bufs[c % 2],
                sems[c % 2],
            )
        )
        if c > 0:
            copies[c - 1].wait()
            pltpu.sync_copy(
                bufs[(c - 1) % 2],
                g_hbm.at[pl.ds(base + (c - 1) * CHUNK, CHUNK)],
            )
    copies[NCHUNK - 1].wait()
    pltpu.sync_copy(
        bufs[(NCHUNK - 1) % 2],
        g_hbm.at[pl.ds(base + (NCHUNK - 1) * CHUNK, CHUNK)],
    )


_gather = functools.partial(
    pl.kernel,
    out_type=jax.ShapeDtypeStruct((SEQ, DIM), jnp.float32),
    mesh=plsc.VectorSubcoreMesh(core_axis_name="c", subcore_axis_name="s"),
    scratch_types=[
        pltpu.VMEM((ROWS_PER_W,), jnp.int32),
        pltpu.VMEM((CHUNK, DIM), jnp.float32),
        pltpu.VMEM((CHUNK, DIM), jnp.float32),
        pltpu.SemaphoreType.DMA,
        pltpu.SemaphoreType.DMA,
    ],
)(_sc_gather_body)


def _sc_add_body(k_hbm, g_hbm, ok_hbm, *scr):
    gbufs = scr[0:DEPTH]
    dbufs = (scr[DEPTH:2 * DEPTH], scr[2 * DEPTH:3 * DEPTH])
    gsems = scr[3 * DEPTH:4 * DEPTH]
    dsems = (scr[4 * DEPTH:5 * DEPTH], scr[5 * DEPTH:6 * DEPTH])
    osems = (scr[6 * DEPTH:7 * DEPTH], scr[7 * DEPTH:8 * DEPTH])

    wid = lax.axis_index("s") * 2 + lax.axis_index("c")
    gbase = wid * (ROWS_PER_W * DIM)

    def g_load(c):
        return pltpu.async_copy(
            g_hbm.at[pl.ds(gbase + c * CE, CE)], gbufs[c % DEPTH], gsems[c % DEPTH]
        )

    def k_load(c, b):
        return pltpu.async_copy(
            k_hbm.at[pl.ds(b * (SEQ * DIM) + gbase + c * CE, CE)],
            dbufs[b][c % DEPTH],
            dsems[b][c % DEPTH],
        )

    g_copies = []
    d_copies = []
    for c in range(DEPTH - 1):
        g_copies.append(g_load(c))
        d_copies.append((k_load(c, 0), k_load(c, 1)))

    for c in range(NACHUNK):
        j = c % DEPTH
        gbuf = gbufs[j]
        d0, d1 = dbufs[0][j], dbufs[1][j]
        g_copies[c].wait()
        d_copies[c][0].wait()
        d_copies[c][1].wait()

        def grp_add(i, carry, d0=d0, d1=d1, gbuf=gbuf):
            for u in range(4):
                s = pl.ds((4 * i + u) * 16, 16)
                g = gbuf[s]
                d0[s] = d0[s] + g
                d1[s] = d1[s] + g
            return carry

        lax.fori_loop(0, CE // 64, grp_add, 0)

        out0 = pltpu.async_copy(
            d0, ok_hbm.at[pl.ds(gbase + c * CE, CE)], osems[0][j]
        )
        out1 = pltpu.async_copy(
            d1, ok_hbm.at[pl.ds(SEQ * DIM + gbase + c * CE, CE)], osems[1][j]
        )
        del out0, out1
        if c + DEPTH - 1 < NACHUNK:
            if c >= 1:
                jj = (c - 1) % DEPTH
                pltpu.make_async_copy(
                    dbufs[0][jj], ok_hbm.at[pl.ds(0, CE)], osems[0][jj]
                ).wait()
                pltpu.make_async_copy(
                    dbufs[1][jj], ok_hbm.at[pl.ds(0, CE)], osems[1][jj]
                ).wait()
            g_copies.append(g_load(c + DEPTH - 1))
            d_copies.append((k_load(c + DEPTH - 1, 0), k_load(c + DEPTH - 1, 1)))

    for c in range(NACHUNK - DEPTH, NACHUNK):
        jj = c % DEPTH
        pltpu.make_async_copy(
            dbufs[0][jj], ok_hbm.at[pl.ds(0, CE)], osems[0][jj]
        ).wait()
        pltpu.make_async_copy(
            dbufs[1][jj], ok_hbm.at[pl.ds(0, CE)], osems[1][jj]
        ).wait()


_sc_add = functools.partial(
    pl.kernel,
    out_type=jax.ShapeDtypeStruct((BATCH * SEQ * DIM,), jnp.float32),
    mesh=plsc.VectorSubcoreMesh(core_axis_name="c", subcore_axis_name="s"),
    scratch_types=(
        [pltpu.VMEM((CE,), jnp.float32) for _ in range(3 * DEPTH)]
        + [pltpu.SemaphoreType.DMA for _ in range(5 * DEPTH)]
    ),
)(_sc_add_body)


def _tc_add(q_ref, g_ref, oq_ref):
    oq_ref[...] = q_ref[...] + g_ref[...][None, :, :]


_q_add = pl.pallas_call(
    _tc_add,
    grid=(NBLK,),
    in_specs=[
        pl.BlockSpec((BATCH, BS, DIM), lambda j: (0, j, 0)),
        pl.BlockSpec((BS, DIM), lambda j: (j, 0)),
    ],
    out_specs=pl.BlockSpec((BATCH, BS, DIM), lambda j: (0, j, 0)),
    out_shape=jax.ShapeDtypeStruct((BATCH, SEQ, DIM), jnp.float32),
)


@jax.jit
def kernel(q, k, pos, pe):
    g = _gather(pe, pos)
    ok = _sc_add(k.reshape(-1), g.reshape(-1))
    oq = _q_add(q, g)
    return oq, ok.reshape(BATCH, SEQ, DIM)


# speedup vs baseline: 1.9147x; 1.9147x over previous
import functools

import jax
import jax.numpy as jnp
from jax import lax
from jax.experimental import pallas as pl
from jax.experimental.pallas import tpu as pltpu
from jax.experimental.pallas import tpu_sc as plsc

DIM = 1024
SEQ = 8192
BATCH = 2

NUM_WORKERS = 32
ROWS_PER_W = SEQ // NUM_WORKERS
CHUNK = 32
NCHUNK = ROWS_PER_W // CHUNK

ACHUNK = 8
NACHUNK = ROWS_PER_W // ACHUNK
CE = ACHUNK * DIM
DEPTH = 4

BS = 512
NBLK = SEQ // BS


def _sc_gather_body(pe_hbm, pos_hbm, g_hbm, idx_v, buf0, buf1, sem0, sem1):
    wid = lax.axis_index("s") * 2 + lax.axis_index("c")
    base = wid * ROWS_PER_W
    pltpu.sync_copy(pos_hbm.at[pl.ds(base, ROWS_PER_W)], idx_v)
    bufs = (buf0, buf1)
    sems = (sem0, sem1)
    copies = []
    for c in range(NCHUNK):
        copies.append(
            pltpu.async_copy(
                pe_hbm.at[idx_v.at[pl.ds(c * CHUNK, CHUNK)]],
                bufs[c % 2],
                sems[c % 2],
            )
        )
        if c > 0:
            copies[c - 1].wait()
            pltpu.sync_copy(
                bufs[(c - 1) % 2],
                g_hbm.at[pl.ds(base + (c - 1) * CHUNK, CHUNK)],
            )
    copies[NCHUNK - 1].wait()
    pltpu.sync_copy(
        bufs[(NCHUNK - 1) % 2],
        g_hbm.at[pl.ds(base + (NCHUNK - 1) * CHUNK, CHUNK)],
    )


_gather = functools.partial(
    pl.kernel,
    out_type=jax.ShapeDtypeStruct((SEQ, DIM), jnp.float32),
    mesh=plsc.VectorSubcoreMesh(core_axis_name="c", subcore_axis_name="s"),
    scratch_types=[
        pltpu.VMEM((ROWS_PER_W,), jnp.int32),
        pltpu.VMEM((CHUNK, DIM), jnp.float32),
        pltpu.VMEM((CHUNK, DIM), jnp.float32),
        pltpu.SemaphoreType.DMA,
        pltpu.SemaphoreType.DMA,
    ],
)(_sc_gather_body)


def _sc_add_body(k_hbm, g_hbm, ok_hbm, *scr):
    gbufs = scr[0:DEPTH]
    dbufs = (scr[DEPTH:2 * DEPTH], scr[2 * DEPTH:3 * DEPTH])
    gsems = scr[3 * DEPTH:4 * DEPTH]
    dsems = (scr[4 * DEPTH:5 * DEPTH], scr[5 * DEPTH:6 * DEPTH])
    osems = (scr[6 * DEPTH:7 * DEPTH], scr[7 * DEPTH:8 * DEPTH])

    wid = lax.axis_index("s") * 2 + lax.axis_index("c")
    gbase = wid * ROWS_PER_W

    def g_load(c):
        return pltpu.async_copy(
            g_hbm.at[pl.ds(gbase + c * ACHUNK, ACHUNK)],
            gbufs[c % DEPTH],
            gsems[c % DEPTH],
        )

    def k_load(c, b):
        return pltpu.async_copy(
            k_hbm.at[pl.ds(b * SEQ + gbase + c * ACHUNK, ACHUNK)],
            dbufs[b][c % DEPTH],
            dsems[b][c % DEPTH],
        )

    g_copies = []
    d_copies = []
    for c in range(DEPTH - 1):
        g_copies.append(g_load(c))
        d_copies.append((k_load(c, 0), k_load(c, 1)))

    for c in range(NACHUNK):
        j = c % DEPTH
        gbuf = gbufs[j]
        d0, d1 = dbufs[0][j], dbufs[1][j]
        g_copies[c].wait()
        d_copies[c][0].wait()
        d_copies[c][1].wait()

        def qrow_add(i, carry, d0=d0, d1=d1, gbuf=gbuf):
            r = i // 4
            q = i % 4
            for u in range(16):
                s = pl.ds(q * 256 + u * 16, 16)
                g = gbuf[r, s]
                d0[r, s] = d0[r, s] + g
                d1[r, s] = d1[r, s] + g
            return carry

        lax.fori_loop(0, ACHUNK * 4, qrow_add, 0)

        pltpu.async_copy(
            d0, ok_hbm.at[pl.ds(gbase + c * ACHUNK, ACHUNK)], osems[0][j]
        )
        pltpu.async_copy(
            d1, ok_hbm.at[pl.ds(SEQ + gbase + c * ACHUNK, ACHUNK)], osems[1][j]
        )
        if c + DEPTH - 1 < NACHUNK:
            if c >= 1:
                jj = (c - 1) % DEPTH
                pltpu.make_async_copy(
                    dbufs[0][jj], ok_hbm.at[pl.ds(0, ACHUNK)], osems[0][jj]
                ).wait()
                pltpu.make_async_copy(
                    dbufs[1][jj], ok_hbm.at[pl.ds(0, ACHUNK)], osems[1][jj]
                ).wait()
            g_copies.append(g_load(c + DEPTH - 1))
            d_copies.append((k_load(c + DEPTH - 1, 0), k_load(c + DEPTH - 1, 1)))

    for c in range(NACHUNK - DEPTH, NACHUNK):
        jj = c % DEPTH
        pltpu.make_async_copy(
            dbufs[0][jj], ok_hbm.at[pl.ds(0, ACHUNK)], osems[0][jj]
        ).wait()
        pltpu.make_async_copy(
            dbufs[1][jj], ok_hbm.at[pl.ds(0, ACHUNK)], osems[1][jj]
        ).wait()


_sc_add = functools.partial(
    pl.kernel,
    out_type=jax.ShapeDtypeStruct((BATCH * SEQ, DIM), jnp.float32),
    mesh=plsc.VectorSubcoreMesh(core_axis_name="c", subcore_axis_name="s"),
    scratch_types=(
        [pltpu.VMEM((ACHUNK, DIM), jnp.float32) for _ in range(3 * DEPTH)]
        + [pltpu.SemaphoreType.DMA for _ in range(5 * DEPTH)]
    ),
)(_sc_add_body)


def _tc_add(q_ref, g_ref, oq_ref):
    oq_ref[...] = q_ref[...] + g_ref[...][None, :, :]


_q_add = pl.pallas_call(
    _tc_add,
    grid=(NBLK,),
    in_specs=[
        pl.BlockSpec((BATCH, BS, DIM), lambda j: (0, j, 0)),
        pl.BlockSpec((BS, DIM), lambda j: (j, 0)),
    ],
    out_specs=pl.BlockSpec((BATCH, BS, DIM), lambda j: (0, j, 0)),
    out_shape=jax.ShapeDtypeStruct((BATCH, SEQ, DIM), jnp.float32),
)


@jax.jit
def kernel(q, k, pos, pe):
    g = _gather(pe, pos)
    ok = _sc_add(k.reshape(BATCH * SEQ, DIM), g)
    oq = _q_add(q, g)
    return oq, ok.reshape(BATCH, SEQ, DIM)


# speedup vs baseline: 2.1843x; 1.1408x over previous
import functools

import jax
import jax.numpy as jnp
from jax import lax
from jax.experimental import pallas as pl
from jax.experimental.pallas import tpu as pltpu
from jax.experimental.pallas import tpu_sc as plsc

DIM = 1024
SEQ = 8192
BATCH = 2

NUM_WORKERS = 32
ROWS_PER_W = SEQ // NUM_WORKERS
CHUNK = 32
NCHUNK = ROWS_PER_W // CHUNK

BS = 256
NBLK = SEQ // BS


def _sc_gather_body(pe_hbm, pos_hbm, g_hbm, idx_v, buf0, buf1, sem0, sem1):
    wid = lax.axis_index("s") * 2 + lax.axis_index("c")
    base = wid * ROWS_PER_W
    pltpu.sync_copy(pos_hbm.at[pl.ds(base, ROWS_PER_W)], idx_v)
    bufs = (buf0, buf1)
    sems = (sem0, sem1)
    copies = []
    for c in range(NCHUNK):
        copies.append(
            pltpu.async_copy(
                pe_hbm.at[idx_v.at[pl.ds(c * CHUNK, CHUNK)]],
                bufs[c % 2],
                sems[c % 2],
            )
        )
        if c > 0:
            copies[c - 1].wait()
            pltpu.sync_copy(
                bufs[(c - 1) % 2],
                g_hbm.at[pl.ds(base + (c - 1) * CHUNK, CHUNK)],
            )
    copies[NCHUNK - 1].wait()
    pltpu.sync_copy(
        bufs[(NCHUNK - 1) % 2],
        g_hbm.at[pl.ds(base + (NCHUNK - 1) * CHUNK, CHUNK)],
    )


_gather = functools.partial(
    pl.kernel,
    out_type=jax.ShapeDtypeStruct((SEQ, DIM), jnp.float32),
    mesh=plsc.VectorSubcoreMesh(core_axis_name="c", subcore_axis_name="s"),
    scratch_types=[
        pltpu.VMEM((ROWS_PER_W,), jnp.int32),
        pltpu.VMEM((CHUNK, DIM), jnp.float32),
        pltpu.VMEM((CHUNK, DIM), jnp.float32),
        pltpu.SemaphoreType.DMA,
        pltpu.SemaphoreType.DMA,
    ],
)(_sc_gather_body)


def _tc_add(q_ref, k_ref, g_ref, oq_ref, ok_ref):
    g = g_ref[...][None, :, :]
    oq_ref[...] = q_ref[...] + g
    ok_ref[...] = k_ref[...] + g


_fused_add = pl.pallas_call(
    _tc_add,
    grid=(NBLK,),
    in_specs=[
        pl.BlockSpec((BATCH, BS, DIM), lambda j: (0, j, 0)),
        pl.BlockSpec((BATCH, BS, DIM), lambda j: (0, j, 0)),
        pl.BlockSpec((BS, DIM), lambda j: (j, 0)),
    ],
    out_specs=[
        pl.BlockSpec((BATCH, BS, DIM), lambda j: (0, j, 0)),
        pl.BlockSpec((BATCH, BS, DIM), lambda j: (0, j, 0)),
    ],
    out_shape=[
        jax.ShapeDtypeStruct((BATCH, SEQ, DIM), jnp.float32),
        jax.ShapeDtypeStruct((BATCH, SEQ, DIM), jnp.float32),
    ],
)


@jax.jit
def kernel(q, k, pos, pe):
    g = _gather(pe, pos)
    oq, ok = _fused_add(q, k, g)
    return oq, ok
